# baseline (speedup 1.0000x reference)
import jax
import jax.numpy as jnp
import numpy as np
from jax import lax
from jax.experimental import pallas as pl
from jax.experimental.pallas import tpu as pltpu, tpu_sc as plsc

_D = 25
_N = 100000
_G = _N // 4
_S1 = 10
_NW = 32
_B1 = 128
_H1 = _B1 * _S1
_CH = 128
_NCH = _H1 // _CH
_CW = _CH * _D
_H2 = _H1 * _D
_TR = 1000


def _pack_tc(adj):
    def body(x, o):
        xr = x[...].reshape(_TR, 4, _D)
        o[...] = jnp.pad(xr, ((0, 0), (0, 0), (0, 7))).reshape(_TR, 128)

    nblk = _G // _TR
    return pl.pallas_call(
        body,
        grid=(nblk,),
        in_specs=[pl.BlockSpec((4 * _TR, _D), lambda i: (i, 0))],
        out_specs=pl.BlockSpec((_TR, 128), lambda i: (i, 0)),
        out_shape=jax.ShapeDtypeStruct((_G, 128), jnp.int32),
    )(adj)


def _sc_sample(padt, seeds, r1, c1, r2, c2):
    mesh = plsc.VectorSubcoreMesh(
        core_axis_name="c", subcore_axis_name="s", num_cores=2, num_subcores=16)

    def body(padt_hbm, seeds_hbm, r1_hbm, c1_hbm, r2_hbm, c2_hbm,
             out1_hbm, out2_hbm,
             ids_v, st1_v, st2a_v, st2b_v, out1_v, out2_v,
             r1_v, c1_v, r2_v, c2_v, g_v, kb_v,
             sem_in, sem_g, sem_w):
        wid = lax.axis_index("s") * 2 + lax.axis_index("c")

        cps = [
            pltpu.async_copy(seeds_hbm.at[pl.ds(wid * _B1, _B1)], ids_v, sem_in),
            pltpu.async_copy(r1_hbm, r1_v, sem_in),
            pltpu.async_copy(c1_hbm, c1_v, sem_in),
            pltpu.async_copy(r2_hbm, r2_v, sem_in),
            pltpu.async_copy(c2_hbm, c2_v, sem_in),
        ]
        for cp in cps:
            cp.wait()

        def decompose(src_ref, n, dst_g, dst_kb, goff):
            @plsc.parallel_loop(0, n, step=16)
            def _(t):
                idv = src_ref[pl.ds(t, 16)]
                dst_g[pl.ds(t, 16)] = (idv >> 2) + goff
                dst_kb[pl.ds(t, 16)] = (idv & 3) * 32

        decompose(ids_v, _B1, g_v, kb_v, 0)
        pltpu.async_copy(padt_hbm.at[g_v.at[pl.ds(0, _B1)]], st1_v, sem_g).wait()

        @plsc.parallel_loop(0, _H1, step=16, unroll=4)
        def _(t):
            rv = r1_v[pl.ds(t, 16)]
            cv = c1_v[pl.ds(t, 16)] + plsc.load_gather(kb_v, [rv])
            out1_v[pl.ds(t, 16)] = plsc.load_gather(st1_v, [rv, cv])

        w1 = pltpu.async_copy(out1_v, out1_hbm.at[pl.ds(wid * _H1, _H1)], sem_w)

        decompose(out1_v, _H1, g_v, kb_v, 0)
        bufs = [st2a_v, st2b_v]
        gs = [None, None]
        gs[0] = pltpu.async_copy(
            padt_hbm.at[g_v.at[pl.ds(0, _CH)]], bufs[0], sem_g)
        for c in range(_NCH):
            b = c % 2
            if c + 1 < _NCH:
                gs[1 - b] = pltpu.async_copy(
                    padt_hbm.at[g_v.at[pl.ds((c + 1) * _CH, _CH)]],
                    bufs[1 - b], sem_g)
            gs[b].wait()
            st = bufs[b]

            @plsc.parallel_loop(0, _CW, step=16, unroll=4)
            def _(t, c=c, st=st):
                rv = r2_v[pl.ds(t, 16)]
                cv = (c2_v[pl.ds(t, 16)]
                      + plsc.load_gather(kb_v, [rv + (c * _CH)]))
                out2_v[pl.ds(c * _CW + t, 16)] = plsc.load_gather(st, [rv, cv])

        w2 = pltpu.async_copy(out2_v, out2_hbm.at[pl.ds(wid * _H2, _H2)], sem_w)
        w1.wait()
        w2.wait()

    run = pl.kernel(
        body,
        out_type=(
            jax.ShapeDtypeStruct((_NW * _H1,), jnp.int32),
            jax.ShapeDtypeStruct((_NW * _H2,), jnp.int32),
        ),
        mesh=mesh,
        compiler_params=pltpu.CompilerParams(
            needs_layout_passes=False, use_tc_tiling_on_sc=True),
        scratch_types=(
            pltpu.VMEM((_B1,), jnp.int32),
            pltpu.VMEM((_B1, 128), jnp.int32),
            pltpu.VMEM((_CH, 128), jnp.int32),
            pltpu.VMEM((_CH, 128), jnp.int32),
            pltpu.VMEM((_H1,), jnp.int32),
            pltpu.VMEM((_H2,), jnp.int32),
            pltpu.VMEM((_H1,), jnp.int32),
            pltpu.VMEM((_H1,), jnp.int32),
            pltpu.VMEM((_CW,), jnp.int32),
            pltpu.VMEM((_CW,), jnp.int32),
            pltpu.VMEM((_H1,), jnp.int32),
            pltpu.VMEM((_H1,), jnp.int32),
            pltpu.SemaphoreType.DMA,
            pltpu.SemaphoreType.DMA,
            pltpu.SemaphoreType.DMA,
        ),
    )
    return run(padt, seeds, r1, c1, r2, c2)


_P1 = np.array([2, 15, 10, 0, 4, 21, 11, 20, 17, 12], dtype=np.int32)
_P2 = np.array([24, 1, 8, 9, 17, 2, 0, 10, 13, 11, 6, 23, 15, 20, 5,
                21, 12, 14, 18, 19, 4, 3, 16, 22, 7], dtype=np.int32)


def kernel(inputs, adj_info):
    padt = _pack_tc(adj_info)
    t1 = jnp.arange(_H1, dtype=jnp.int32)
    r1 = t1 // _S1
    c1 = jnp.asarray(_P1)[t1 % _S1]
    t2 = jnp.arange(_CW, dtype=jnp.int32)
    r2 = t2 // _D
    c2 = jnp.asarray(_P2)[t2 % _D]
    out1, out2 = _sc_sample(padt, inputs, r1, c1, r2, c2)
    support_sizes = jnp.array([1, _S1, _S1 * _D], dtype=jnp.int32)
    return inputs, out1, out2, support_sizes


# speedup vs baseline: 1.4947x; 1.4947x over previous
import jax
import jax.numpy as jnp
import numpy as np
from jax import lax
from jax.experimental import pallas as pl
from jax.experimental.pallas import tpu as pltpu, tpu_sc as plsc

_D = 25
_N = 100000
_G = _N // 4
_S1 = 10
_NW = 32
_B1 = 128
_H1 = _B1 * _S1
_CH = 128
_NCH = _H1 // _CH
_CW = _CH * _D
_H2 = _H1 * _D
_TR = 1000


def _pack_tc(adj):
    adj_t = jnp.swapaxes(adj, 0, 1)

    bn = 16384

    def body(x, o):
        xt = x[...].T
        xr = xt.reshape(bn // 4, 4, _D)
        o[...] = jnp.pad(xr, ((0, 0), (0, 0), (0, 7))).reshape(bn // 4, 128)

    nblk = (_N + bn - 1) // bn
    return pl.pallas_call(
        body,
        grid=(nblk,),
        in_specs=[pl.BlockSpec((_D, bn), lambda i: (0, i))],
        out_specs=pl.BlockSpec((bn // 4, 128), lambda i: (i, 0)),
        out_shape=jax.ShapeDtypeStruct((_G, 128), jnp.int32),
    )(adj_t)


def _sc_sample(padt, seeds, r1, c1, r2, c2):
    mesh = plsc.VectorSubcoreMesh(
        core_axis_name="c", subcore_axis_name="s", num_cores=2, num_subcores=16)

    def body(padt_hbm, seeds_hbm, r1_hbm, c1_hbm, r2_hbm, c2_hbm,
             out1_hbm, out2_hbm,
             ids_v, st1_v, st2a_v, st2b_v, out1_v, out2_v,
             r1_v, c1_v, r2_v, c2_v, g_v, kb_v,
             sem_in, sem_g, sem_w):
        wid = lax.axis_index("s") * 2 + lax.axis_index("c")

        cps = [
            pltpu.async_copy(seeds_hbm.at[pl.ds(wid * _B1, _B1)], ids_v, sem_in),
            pltpu.async_copy(r1_hbm, r1_v, sem_in),
            pltpu.async_copy(c1_hbm, c1_v, sem_in),
            pltpu.async_copy(r2_hbm, r2_v, sem_in),
            pltpu.async_copy(c2_hbm, c2_v, sem_in),
        ]
        for cp in cps:
            cp.wait()

        def decompose(src_ref, n, dst_g, dst_kb, goff):
            @plsc.parallel_loop(0, n, step=16)
            def _(t):
                idv = src_ref[pl.ds(t, 16)]
                dst_g[pl.ds(t, 16)] = (idv >> 2) + goff
                dst_kb[pl.ds(t, 16)] = (idv & 3) * 32

        decompose(ids_v, _B1, g_v, kb_v, 0)
        pltpu.async_copy(padt_hbm.at[g_v.at[pl.ds(0, _B1)]], st1_v, sem_g).wait()

        @plsc.parallel_loop(0, _H1, step=16, unroll=4)
        def _(t):
            rv = r1_v[pl.ds(t, 16)]
            cv = c1_v[pl.ds(t, 16)] + plsc.load_gather(kb_v, [rv])
            out1_v[pl.ds(t, 16)] = plsc.load_gather(st1_v, [rv, cv])

        w1 = pltpu.async_copy(out1_v, out1_hbm.at[pl.ds(wid * _H1, _H1)], sem_w)

        decompose(out1_v, _H1, g_v, kb_v, 0)
        bufs = [st2a_v, st2b_v]
        gs = [None, None]
        gs[0] = pltpu.async_copy(
            padt_hbm.at[g_v.at[pl.ds(0, _CH)]], bufs[0], sem_g)
        for c in range(_NCH):
            b = c % 2
            if c + 1 < _NCH:
                gs[1 - b] = pltpu.async_copy(
                    padt_hbm.at[g_v.at[pl.ds((c + 1) * _CH, _CH)]],
                    bufs[1 - b], sem_g)
            gs[b].wait()
            st = bufs[b]

            @plsc.parallel_loop(0, _CW, step=16, unroll=4)
            def _(t, c=c, st=st):
                rv = r2_v[pl.ds(t, 16)]
                cv = (c2_v[pl.ds(t, 16)]
                      + plsc.load_gather(kb_v, [rv + (c * _CH)]))
                out2_v[pl.ds(c * _CW + t, 16)] = plsc.load_gather(st, [rv, cv])

        w2 = pltpu.async_copy(out2_v, out2_hbm.at[pl.ds(wid * _H2, _H2)], sem_w)
        w1.wait()
        w2.wait()

    run = pl.kernel(
        body,
        out_type=(
            jax.ShapeDtypeStruct((_NW * _H1,), jnp.int32),
            jax.ShapeDtypeStruct((_NW * _H2,), jnp.int32),
        ),
        mesh=mesh,
        compiler_params=pltpu.CompilerParams(
            needs_layout_passes=False, use_tc_tiling_on_sc=True),
        scratch_types=(
            pltpu.VMEM((_B1,), jnp.int32),
            pltpu.VMEM((_B1, 128), jnp.int32),
            pltpu.VMEM((_CH, 128), jnp.int32),
            pltpu.VMEM((_CH, 128), jnp.int32),
            pltpu.VMEM((_H1,), jnp.int32),
            pltpu.VMEM((_H2,), jnp.int32),
            pltpu.VMEM((_H1,), jnp.int32),
            pltpu.VMEM((_H1,), jnp.int32),
            pltpu.VMEM((_CW,), jnp.int32),
            pltpu.VMEM((_CW,), jnp.int32),
            pltpu.VMEM((_H1,), jnp.int32),
            pltpu.VMEM((_H1,), jnp.int32),
            pltpu.SemaphoreType.DMA,
            pltpu.SemaphoreType.DMA,
            pltpu.SemaphoreType.DMA,
        ),
    )
    return run(padt, seeds, r1, c1, r2, c2)


_P1 = np.array([2, 15, 10, 0, 4, 21, 11, 20, 17, 12], dtype=np.int32)
_P2 = np.array([24, 1, 8, 9, 17, 2, 0, 10, 13, 11, 6, 23, 15, 20, 5,
                21, 12, 14, 18, 19, 4, 3, 16, 22, 7], dtype=np.int32)


def kernel(inputs, adj_info):
    padt = _pack_tc(adj_info)
    t1 = jnp.arange(_H1, dtype=jnp.int32)
    r1 = t1 // _S1
    c1 = jnp.asarray(_P1)[t1 % _S1]
    t2 = jnp.arange(_CW, dtype=jnp.int32)
    r2 = t2 // _D
    c2 = jnp.asarray(_P2)[t2 % _D]
    out1, out2 = _sc_sample(padt, inputs, r1, c1, r2, c2)
    support_sizes = jnp.array([1, _S1, _S1 * _D], dtype=jnp.int32)
    return inputs, out1, out2, support_sizes


# speedup vs baseline: 1.5904x; 1.0641x over previous
import jax
import jax.numpy as jnp
import numpy as np
from jax import lax
from jax.experimental import pallas as pl
from jax.experimental.pallas import tpu as pltpu, tpu_sc as plsc

_D = 25
_N = 100000
_G = _N // 4
_S1 = 10
_NW = 32
_B1 = 128
_H1 = _B1 * _S1
_CH = 128
_NCH = _H1 // _CH
_CW = _CH * _D
_H2 = _H1 * _D
_TR = 1000


def _pack_tc(adj):
    adj_t = jnp.swapaxes(adj, 0, 1)

    bn = 8192

    def body(x, o):
        xt = x[...].T
        xr = xt.reshape(bn // 4, 4, _D)
        o[...] = jnp.pad(xr, ((0, 0), (0, 0), (0, 7))).reshape(bn // 4, 128)

    nblk = (_N + bn - 1) // bn
    return pl.pallas_call(
        body,
        grid=(nblk,),
        in_specs=[pl.BlockSpec((_D, bn), lambda i: (0, i))],
        out_specs=pl.BlockSpec((bn // 4, 128), lambda i: (i, 0)),
        out_shape=jax.ShapeDtypeStruct((_G, 128), jnp.int32),
    )(adj_t)


def _sc_sample(padt, seeds, r1, c1, r2, c2):
    mesh = plsc.VectorSubcoreMesh(
        core_axis_name="c", subcore_axis_name="s", num_cores=2, num_subcores=16)

    def body(padt_hbm, seeds_hbm, r1_hbm, c1_hbm, r2_hbm, c2_hbm,
             out1_hbm, out2_hbm,
             ids_v, st1_v, st2a_v, st2b_v, st2c_v, out1_v, out2_v,
             r1_v, c1_v, r2_v, c2_v, g_v, kb_v,
             sem_in, sem_g, sem_w):
        wid = lax.axis_index("s") * 2 + lax.axis_index("c")

        cps = [
            pltpu.async_copy(seeds_hbm.at[pl.ds(wid * _B1, _B1)], ids_v, sem_in),
            pltpu.async_copy(r1_hbm, r1_v, sem_in),
            pltpu.async_copy(c1_hbm, c1_v, sem_in),
            pltpu.async_copy(r2_hbm, r2_v, sem_in),
            pltpu.async_copy(c2_hbm, c2_v, sem_in),
        ]
        for cp in cps:
            cp.wait()

        def decompose(src_ref, n, dst_g, dst_kb, goff):
            @plsc.parallel_loop(0, n, step=16)
            def _(t):
                idv = src_ref[pl.ds(t, 16)]
                dst_g[pl.ds(t, 16)] = (idv >> 2) + goff
                dst_kb[pl.ds(t, 16)] = (idv & 3) * 32

        decompose(ids_v, _B1, g_v, kb_v, 0)
        pltpu.async_copy(padt_hbm.at[g_v.at[pl.ds(0, _B1)]], st1_v, sem_g).wait()

        @plsc.parallel_loop(0, _H1, step=16, unroll=4)
        def _(t):
            rv = r1_v[pl.ds(t, 16)]
            cv = c1_v[pl.ds(t, 16)] + plsc.load_gather(kb_v, [rv])
            out1_v[pl.ds(t, 16)] = plsc.load_gather(st1_v, [rv, cv])

        w1 = pltpu.async_copy(out1_v, out1_hbm.at[pl.ds(wid * _H1, _H1)], sem_w)

        decompose(out1_v, _H1, g_v, kb_v, 0)
        bufs = [st2a_v, st2b_v, st2c_v]
        gs = [None, None, None]
        for c in range(2):
            gs[c] = pltpu.async_copy(
                padt_hbm.at[g_v.at[pl.ds(c * _CH, _CH)]], bufs[c], sem_g)
        for c in range(_NCH):
            b = c % 3
            if c + 2 < _NCH:
                gs[(c + 2) % 3] = pltpu.async_copy(
                    padt_hbm.at[g_v.at[pl.ds((c + 2) * _CH, _CH)]],
                    bufs[(c + 2) % 3], sem_g)
            gs[b].wait()
            st = bufs[b]

            @plsc.parallel_loop(0, _CW, step=16, unroll=4)
            def _(t, c=c, st=st):
                rv = r2_v[pl.ds(t, 16)]
                cv = (c2_v[pl.ds(t, 16)]
                      + plsc.load_gather(kb_v, [rv + (c * _CH)]))
                out2_v[pl.ds(c * _CW + t, 16)] = plsc.load_gather(st, [rv, cv])

        w2 = pltpu.async_copy(out2_v, out2_hbm.at[pl.ds(wid * _H2, _H2)], sem_w)
        w1.wait()
        w2.wait()

    run = pl.kernel(
        body,
        out_type=(
            jax.ShapeDtypeStruct((_NW * _H1,), jnp.int32),
            jax.ShapeDtypeStruct((_NW * _H2,), jnp.int32),
        ),
        mesh=mesh,
        compiler_params=pltpu.CompilerParams(
            needs_layout_passes=False, use_tc_tiling_on_sc=True),
        scratch_types=(
            pltpu.VMEM((_B1,), jnp.int32),
            pltpu.VMEM((_B1, 128), jnp.int32),
            pltpu.VMEM((_CH, 128), jnp.int32),
            pltpu.VMEM((_CH, 128), jnp.int32),
            pltpu.VMEM((_CH, 128), jnp.int32),
            pltpu.VMEM((_H1,), jnp.int32),
            pltpu.VMEM((_H2,), jnp.int32),
            pltpu.VMEM((_H1,), jnp.int32),
            pltpu.VMEM((_H1,), jnp.int32),
            pltpu.VMEM((_CW,), jnp.int32),
            pltpu.VMEM((_CW,), jnp.int32),
            pltpu.VMEM((_H1,), jnp.int32),
            pltpu.VMEM((_H1,), jnp.int32),
            pltpu.SemaphoreType.DMA,
            pltpu.SemaphoreType.DMA,
            pltpu.SemaphoreType.DMA,
        ),
    )
    return run(padt, seeds, r1, c1, r2, c2)


_P1 = np.array([2, 15, 10, 0, 4, 21, 11, 20, 17, 12], dtype=np.int32)
_P2 = np.array([24, 1, 8, 9, 17, 2, 0, 10, 13, 11, 6, 23, 15, 20, 5,
                21, 12, 14, 18, 19, 4, 3, 16, 22, 7], dtype=np.int32)


def kernel(inputs, adj_info):
    padt = _pack_tc(adj_info)
    t1 = jnp.arange(_H1, dtype=jnp.int32)
    r1 = t1 // _S1
    c1 = jnp.asarray(_P1)[t1 % _S1]
    t2 = jnp.arange(_CW, dtype=jnp.int32)
    r2 = t2 // _D
    c2 = jnp.asarray(_P2)[t2 % _D]
    out1, out2 = _sc_sample(padt, inputs, r1, c1, r2, c2)
    support_sizes = jnp.array([1, _S1, _S1 * _D], dtype=jnp.int32)
    return inputs, out1, out2, support_sizes
